# baseline (speedup 1.0000x reference)
import functools

import jax
import jax.numpy as jnp
from jax import lax
from jax.experimental import pallas as pl
from jax.experimental.pallas import tpu as pltpu
from jax.experimental.pallas import tpu_sc as plsc

B = 4096
LF = 50
LP = 20
D = 64
NROWS = 2 + LP + LF

_info = plsc.get_sparse_core_info()
NC, NS, L = _info.num_cores, _info.num_subcores, _info.num_lanes
NW = NC * NS
PER_W = B // NW
CE = 8
NCH = PER_W // CE
CROWS = CE * NROWS

_mesh = plsc.VectorSubcoreMesh(core_axis_name="c", subcore_axis_name="s")


def _fm_body(uiidx_h, pfidx_h, bias_h, ui_table_h, feat_table_h,
             res_h, biasm_h, rows_h,
             idx_ui_v, idx_pf_v, rows_all, biasst, res_buf, bias_v, sem):
    wid = lax.axis_index("s") * NC + lax.axis_index("c")
    pltpu.sync_copy(bias_h, bias_v)
    bvec = bias_v[...]
    lane = lax.iota(jnp.int32, L)
    lane0 = lane == 0
    zero = jnp.zeros((L,), jnp.float32)

    def chunk_body(ch, carry):
        base = wid * PER_W + ch * CE
        pltpu.sync_copy(uiidx_h.at[pl.ds(base, CE), :], idx_ui_v)
        pltpu.sync_copy(pfidx_h.at[pl.ds(base, CE), :], idx_pf_v)

        cps = []
        for e in range(CE):
            r0 = e * NROWS
            cps.append(pltpu.async_copy(
                ui_table_h.at[idx_ui_v.at[e]],
                rows_all.at[pl.ds(r0, 2), :], sem))
            cps.append(pltpu.async_copy(
                feat_table_h.at[idx_pf_v.at[e]],
                rows_all.at[pl.ds(r0 + 2, LP + LF), :], sem))
        for cp in cps:
            cp.wait()

        for e in range(CE):
            r0 = e * NROWS

            def fbody(r, fs):
                row = r0 + 2 + LP + r * 5
                out = list(fs)
                for j in range(5):
                    for c in range(4):
                        out[c] = out[c] + rows_all[row + j, pl.ds(c * L, L)]
                return tuple(out)
            f = lax.fori_loop(0, LF // 5, fbody, (zero,) * 4)

            def pbody(r, c8):
                row = r0 + 2 + r * 4
                p = list(c8[:4])
                q = list(c8[4:])
                for j in range(4):
                    for c in range(4):
                        v = rows_all[row + j, pl.ds(c * L, L)]
                        p[c] = p[c] + v
                        q[c] = q[c] + v * v
                return tuple(p) + tuple(q)
            pq = lax.fori_loop(0, LP // 4, pbody, (zero,) * 8)

            acc = zero
            for c in range(4):
                u0 = rows_all[r0, pl.ds(c * L, L)]
                u1 = rows_all[r0 + 1, pl.ds(c * L, L)]
                usum = u0 + u1
                usq = u0 * u0 + u1 * u1
                s = usum + pq[c]
                acc = acc + (0.5 * (s * s) + f[c] * pq[c]
                             - 0.5 * usq - 0.5 * pq[4 + c])
            for sh in (8, 4, 2, 1):
                acc = acc + _permute(acc, lane ^ sh)
            plsc.store_scatter(res_buf,
                               [jnp.full((L,), ch * CE + e, jnp.int32)],
                               acc + bvec, mask=lane0)

        col_bias = jnp.full((L,), D, jnp.int32)
        ui_row = (lane >> 1) * NROWS + (lane & 1)
        g = plsc.load_gather(rows_all, [ui_row, col_bias])
        pos = (lane >> 1) * (LP + 2) + (lane & 1)
        plsc.store_scatter(biasst, [pos], g)
        for k in range(CE * LP // L):
            gidx = k * L + lane
            ee = gidx // LP
            rr = gidx - ee * LP
            val = plsc.load_gather(rows_all,
                                   [ee * NROWS + 2 + rr, col_bias])
            plsc.store_scatter(biasst, [ee * (LP + 2) + 2 + rr], val)
        pltpu.sync_copy(
            biasst,
            biasm_h.at[pl.ds(pl.multiple_of(base * (LP + 2), 8), CE * (LP + 2))])

        pltpu.sync_copy(
            rows_all.at[:, 0:D],
            rows_h.at[pl.ds(pl.multiple_of(base * NROWS, 8), CROWS), :])
        return carry

    lax.fori_loop(0, NCH, chunk_body, 0)
    pltpu.sync_copy(res_buf,
                    res_h.at[pl.ds(pl.multiple_of(wid * PER_W, 8), PER_W)])


_GDN = lax.GatherDimensionNumbers(
    offset_dims=(), collapsed_slice_dims=(0,), start_index_map=(0,))


def _permute(v, idx):
    return lax.gather(v, idx[:, None], _GDN, (1,),
                      mode=lax.GatherScatterMode.PROMISE_IN_BOUNDS)


_fm_call = functools.partial(
    pl.kernel,
    mesh=_mesh,
    compiler_params=pltpu.CompilerParams(needs_layout_passes=False,
                                         use_tc_tiling_on_sc=False),
    out_type=[
        jax.ShapeDtypeStruct((B,), jnp.float32),
        jax.ShapeDtypeStruct((B * (LP + 2),), jnp.float32),
        jax.ShapeDtypeStruct((B * NROWS, D), jnp.float32),
    ],
    scratch_types=[
        pltpu.VMEM((CE, 2), jnp.int32),
        pltpu.VMEM((CE, 70), jnp.int32),
        pltpu.VMEM((CROWS, 72), jnp.float32),
        pltpu.VMEM((CE * (LP + 2),), jnp.float32),
        pltpu.VMEM((PER_W,), jnp.float32),
        pltpu.VMEM((L,), jnp.float32),
        pltpu.SemaphoreType.DMA,
    ],
)(_fm_body)


def kernel(ui_pair, feature_index, preference_index, ui_table, feat_table, Bias):
    uiidx = ui_pair.astype(jnp.int32)
    pfidx = jnp.concatenate(
        [preference_index.astype(jnp.int32), feature_index.astype(jnp.int32)],
        axis=1)
    bias16 = jnp.broadcast_to(Bias.astype(jnp.float32), (L,))
    uip = jnp.pad(ui_table, ((0, 0), (0, 72 - 65)))
    ftp = jnp.pad(feat_table, ((0, 0), (0, 72 - 65)))
    res, biasm, rows = _fm_call(uiidx, pfidx, bias16, uip, ftp)
    return (res.reshape(B, 1),
            biasm.reshape(B, LP + 2, 1),
            rows.reshape(B, NROWS, D))


# speedup vs baseline: 1.2703x; 1.2703x over previous
import functools

import jax
import jax.numpy as jnp
from jax import lax
from jax.experimental import pallas as pl
from jax.experimental.pallas import tpu as pltpu
from jax.experimental.pallas import tpu_sc as plsc

B = 4096
LF = 50
LP = 20
D = 64
NROWS = 2 + LP + LF

_info = plsc.get_sparse_core_info()
NC, NS, L = _info.num_cores, _info.num_subcores, _info.num_lanes
NW = NC * NS
PER_W = B // NW
CE = 8
NCH = PER_W // CE
CROWS = CE * NROWS

_mesh = plsc.VectorSubcoreMesh(core_axis_name="c", subcore_axis_name="s")


def _fm_body(uiidx_h, pfidx_h, bias_h, ui_table_h, feat_table_h,
             res_h, biasm_h, rows_h,
             idx_ui_v, idx_pf_v, rows_all, biasst, res_buf, bias_v, sem):
    wid = lax.axis_index("s") * NC + lax.axis_index("c")
    pltpu.sync_copy(bias_h, bias_v)
    bvec = bias_v[...]
    lane = lax.iota(jnp.int32, L)
    lane0 = lane == 0
    zero = jnp.zeros((L,), jnp.float32)

    def chunk_body(ch, carry):
        base = wid * PER_W + ch * CE
        pltpu.sync_copy(uiidx_h.at[pl.ds(base, CE), :], idx_ui_v)
        pltpu.sync_copy(pfidx_h.at[pl.ds(base, CE), :], idx_pf_v)

        cps = []
        for e in range(CE):
            r0 = e * NROWS
            cps.append(pltpu.async_copy(
                ui_table_h.at[idx_ui_v.at[e]],
                rows_all.at[pl.ds(r0, 2), :], sem))
            cps.append(pltpu.async_copy(
                feat_table_h.at[idx_pf_v.at[e]],
                rows_all.at[pl.ds(r0 + 2, LP + LF), :], sem))
        for cp in cps:
            cp.wait()

        for e in range(CE):
            r0 = e * NROWS

            def fbody(r, fs):
                row = r0 + 2 + LP + r * 5
                out = list(fs)
                for j in range(5):
                    for c in range(4):
                        out[c] = out[c] + rows_all[row + j, pl.ds(c * L, L)]
                return tuple(out)
            f = lax.fori_loop(0, LF // 5, fbody, (zero,) * 4)

            def pbody(r, c8):
                row = r0 + 2 + r * 4
                p = list(c8[:4])
                q = list(c8[4:])
                for j in range(4):
                    for c in range(4):
                        v = rows_all[row + j, pl.ds(c * L, L)]
                        p[c] = p[c] + v
                        q[c] = q[c] + v * v
                return tuple(p) + tuple(q)
            pq = lax.fori_loop(0, LP // 4, pbody, (zero,) * 8)

            acc = zero
            for c in range(4):
                u0 = rows_all[r0, pl.ds(c * L, L)]
                u1 = rows_all[r0 + 1, pl.ds(c * L, L)]
                usum = u0 + u1
                usq = u0 * u0 + u1 * u1
                s = usum + pq[c]
                acc = acc + (0.5 * (s * s) + f[c] * pq[c]
                             - 0.5 * usq - 0.5 * pq[4 + c])
            for sh in (8, 4, 2, 1):
                acc = acc + _permute(acc, lane ^ sh)
            plsc.store_scatter(res_buf,
                               [jnp.full((L,), ch * CE + e, jnp.int32)],
                               acc + bvec, mask=lane0)

        col_bias = jnp.full((L,), D, jnp.int32)
        ui_row = (lane >> 1) * NROWS + (lane & 1)
        g = plsc.load_gather(rows_all, [ui_row, col_bias])
        pos = (lane >> 1) * (LP + 2) + (lane & 1)
        plsc.store_scatter(biasst, [pos], g)
        for k in range(CE * LP // L):
            gidx = k * L + lane
            ee = gidx // LP
            rr = gidx - ee * LP
            val = plsc.load_gather(rows_all,
                                   [ee * NROWS + 2 + rr, col_bias])
            plsc.store_scatter(biasst, [ee * (LP + 2) + 2 + rr], val)
        pltpu.sync_copy(
            biasst,
            biasm_h.at[pl.ds(pl.multiple_of(base * (LP + 2), 8), CE * (LP + 2))])

        pltpu.sync_copy(
            rows_all.at[:, 0:D],
            rows_h.at[pl.ds(pl.multiple_of(base * NROWS, 8), CROWS), :])
        return carry

    lax.fori_loop(0, NCH, chunk_body, 0)
    pltpu.sync_copy(res_buf,
                    res_h.at[pl.ds(pl.multiple_of(wid * PER_W, 8), PER_W)])


_GDN = lax.GatherDimensionNumbers(
    offset_dims=(), collapsed_slice_dims=(0,), start_index_map=(0,))


def _permute(v, idx):
    return lax.gather(v, idx[:, None], _GDN, (1,),
                      mode=lax.GatherScatterMode.PROMISE_IN_BOUNDS)


_fm_call = functools.partial(
    pl.kernel,
    mesh=_mesh,
    compiler_params=pltpu.CompilerParams(needs_layout_passes=False,
                                         use_tc_tiling_on_sc=False),
    out_type=[
        jax.ShapeDtypeStruct((B,), jnp.float32),
        jax.ShapeDtypeStruct((B * (LP + 2),), jnp.float32),
        jax.ShapeDtypeStruct((B * NROWS, D), jnp.float32),
    ],
    scratch_types=[
        pltpu.VMEM((CE, 2), jnp.int32),
        pltpu.VMEM((CE, 70), jnp.int32),
        pltpu.VMEM((CROWS, 128), jnp.float32),
        pltpu.VMEM((CE * (LP + 2),), jnp.float32),
        pltpu.VMEM((PER_W,), jnp.float32),
        pltpu.VMEM((L,), jnp.float32),
        pltpu.SemaphoreType.DMA,
    ],
)(_fm_body)


def kernel(ui_pair, feature_index, preference_index, ui_table, feat_table, Bias):
    uiidx = ui_pair.astype(jnp.int32)
    pfidx = jnp.concatenate(
        [preference_index.astype(jnp.int32), feature_index.astype(jnp.int32)],
        axis=1)
    bias16 = jnp.broadcast_to(Bias.astype(jnp.float32), (L,))
    uip = jnp.pad(ui_table, ((0, 0), (0, 128 - 65)))
    ftp = jnp.pad(feat_table, ((0, 0), (0, 128 - 65)))
    res, biasm, rows = _fm_call(uiidx, pfidx, bias16, uip, ftp)
    return (res.reshape(B, 1),
            biasm.reshape(B, LP + 2, 1),
            rows.reshape(B, NROWS, D))
